# baseline (speedup 1.0000x reference)
import functools

import jax
import jax.numpy as jnp
from jax import lax
from jax.experimental import pallas as pl
from jax.experimental.pallas import tpu as pltpu
from jax.experimental.pallas import tpu_sc as plsc

_N = 10000
_NPAD = 10240
_E = 320000
_EPAD = 327680
_NAGG = 10016
_H = 128
_HH = 64
_G = 64
_CH = 128
_LANE = 16
_NCHUNKS_E = _EPAD // _CH
_TCH_E = _NCHUNKS_E // 16

_mesh = plsc.VectorSubcoreMesh(core_axis_name="c", subcore_axis_name="s")
_sc_params = pltpu.CompilerParams(use_tc_tiling_on_sc=False)


@functools.partial(pl.kernel,
                   out_type=jax.ShapeDtypeStruct((2 * _NPAD, _HH), jnp.float32),
                   mesh=_mesh,
                   compiler_params=_sc_params,
                   scratch_types=[
                       pltpu.VMEM((9 * _CH,), jnp.int32),
                       pltpu.VMEM((2, _CH, _HH), jnp.float32),
                       pltpu.VMEM((_CH, _HH), jnp.float32),
                       pltpu.SemaphoreType.DMA,
                       pltpu.SemaphoreType.DMA,
                   ])
def _sc_atom_encoder(xP_hbm, tab_hbm, h0_hbm, ibuf, rows, acc, g0, g1):
    c = lax.axis_index("c")
    s = lax.axis_index("s")
    sems = (g0, g1)

    @pl.loop(0, _NPAD // _CH // 16)
    def _chunk(t):
        gc = s + 16 * t
        pltpu.sync_copy(xP_hbm.at[pl.ds(gc * 9 * _CH, 9 * _CH)], ibuf)

        @plsc.parallel_loop(0, 9 * _CH, _LANE)
        def _off(i):
            ibuf[pl.ds(i, _LANE)] = (ibuf[pl.ds(i, _LANE)]
                                     + c * (9 * _H) + (i // _CH) * _H)

        pltpu.sync_copy(tab_hbm.at[ibuf.at[pl.ds(0, _CH)]], rows.at[0])
        for f in range(1, 9):
            b = f % 2
            pltpu.sync_copy(tab_hbm.at[ibuf.at[pl.ds(f * _CH, _CH)]],
                            rows.at[b])
            pb = (f - 1) % 2
            if f == 1:
                @plsc.parallel_loop(0, _CH)
                def _cp(r):
                    for j in range(0, _HH, _LANE):
                        acc[r, pl.ds(j, _LANE)] = rows[pb, r, pl.ds(j, _LANE)]
            else:
                @plsc.parallel_loop(0, _CH)
                def _add(r):
                    for j in range(0, _HH, _LANE):
                        acc[r, pl.ds(j, _LANE)] = (
                            acc[r, pl.ds(j, _LANE)]
                            + rows[pb, r, pl.ds(j, _LANE)])
        @plsc.parallel_loop(0, _CH)
        def _add8(r):
            for j in range(0, _HH, _LANE):
                acc[r, pl.ds(j, _LANE)] = (
                    acc[r, pl.ds(j, _LANE)] + rows[0, r, pl.ds(j, _LANE)])

        pltpu.sync_copy(acc, h0_hbm.at[pl.ds(c * _NPAD + gc * _CH, _CH)])


@functools.partial(pl.kernel,
                   out_type=jax.ShapeDtypeStruct((2 * _N, _HH), jnp.float32),
                   mesh=_mesh,
                   compiler_params=_sc_params,
                   scratch_types=[
                       pltpu.VMEM((2, 4 * _CH), jnp.int32),
                       pltpu.VMEM((2, _CH), jnp.int32),
                       pltpu.VMEM((2, _CH), jnp.int32),
                       pltpu.VMEM((3 * 8 * _HH,), jnp.float32),
                       pltpu.VMEM((2, _CH, _HH), jnp.float32),
                       pltpu.VMEM((2, _CH, _HH), jnp.float32),
                       pltpu.VMEM((2, _CH, _HH), jnp.float32),
                       pltpu.VMEM_SHARED((_NAGG, _HH), jnp.float32),
                       pltpu.VMEM_SHARED((512, _HH), jnp.float32),
                       pltpu.SemaphoreType.DMA,
                       pltpu.SemaphoreType.DMA,
                       pltpu.SemaphoreType.DMA,
                       pltpu.SemaphoreType.DMA,
                       pltpu.SemaphoreType.DMA,
                       pltpu.SemaphoreType.DMA,
                   ])
def _sc_edge_stage(h_hbm, ip_hbm, dp_hbm, be_hbm, out_hbm,
                   ipack, dstb, codeb, bflat, hbuf, ebuf, mbuf,
                   agg_sh, etab_sh, is0, is1, gs0, gs1, ss0, ss1):
    c = lax.axis_index("c")
    s = lax.axis_index("s")
    isem = (is0, is1)
    gsem = (gs0, gs1)
    ssem = (ss0, ss1)

    pltpu.sync_copy(be_hbm.at[pl.ds(c * (24 * _HH), 24 * _HH)], bflat)
    for rl in range(32):
        r = s * 32 + rl
        c0 = r // 64
        c1 = (r // 8) % 8
        c2 = r % 8
        for j in range(0, _HH, _LANE):
            ebuf[0, rl, pl.ds(j, _LANE)] = (
                bflat[pl.ds(c0 * _HH + j, _LANE)]
                + bflat[pl.ds(8 * _HH + c1 * _HH + j, _LANE)]
                + bflat[pl.ds(16 * _HH + c2 * _HH + j, _LANE)])
    pltpu.sync_copy(ebuf.at[0, pl.ds(0, 32)], etab_sh.at[pl.ds(s * 32, 32)])

    @plsc.parallel_loop(0, _CH)
    def _zrow(r):
        for j in range(0, _HH, _LANE):
            mbuf[0, r, pl.ds(j, _LANE)] = jnp.zeros((_LANE,), jnp.float32)

    nblk = 4 + jnp.where(s < 14, 1, 0)

    @pl.loop(0, nblk)
    def _zblk(k):
        pltpu.sync_copy(mbuf.at[0], agg_sh.at[pl.ds((s + 16 * k) * _CH, _CH)])

    @pl.when(s == 0)
    def _ztail():
        pltpu.sync_copy(mbuf.at[0, pl.ds(0, 32)],
                        agg_sh.at[pl.ds(78 * _CH, 32)])

    plsc.subcore_barrier()

    def issue_idx(t, b):
        gc = s + 16 * t
        pltpu.async_copy(ip_hbm.at[pl.ds(gc * 4 * _CH, 4 * _CH)],
                         ipack.at[b], isem[b])
        pltpu.async_copy(dp_hbm.at[pl.ds(gc * _CH, _CH)], dstb.at[b], isem[b])

    def wait_idx(t, b):
        gc = s + 16 * t
        pltpu.make_async_copy(ip_hbm.at[pl.ds(gc * 4 * _CH, 4 * _CH)],
                              ipack.at[b], isem[b]).wait()
        pltpu.make_async_copy(dp_hbm.at[pl.ds(gc * _CH, _CH)], dstb.at[b],
                              isem[b]).wait()

    def compute_code(b):
        @plsc.parallel_loop(0, _CH, _LANE)
        def _srcoff(i):
            ipack[b, pl.ds(i, _LANE)] = ipack[b, pl.ds(i, _LANE)] + c * _N

        @plsc.parallel_loop(0, _CH, _LANE)
        def _code(i):
            codeb[b, pl.ds(i, _LANE)] = (
                ipack[b, pl.ds(_CH + i, _LANE)] * 64
                + ipack[b, pl.ds(2 * _CH + i, _LANE)] * 8
                + ipack[b, pl.ds(3 * _CH + i, _LANE)])

    def issue_gathers(b):
        pltpu.async_copy(h_hbm.at[ipack.at[b, pl.ds(0, _CH)]], hbuf.at[b],
                         gsem[b])
        pltpu.async_copy(etab_sh.at[codeb.at[b]], ebuf.at[b], gsem[b])

    def wait_gathers(b):
        pltpu.make_async_copy(h_hbm.at[ipack.at[b, pl.ds(0, _CH)]],
                              hbuf.at[b], gsem[b]).wait()
        pltpu.make_async_copy(etab_sh.at[codeb.at[b]], ebuf.at[b],
                              gsem[b]).wait()

    def relu_msg(b):
        @plsc.parallel_loop(0, _CH, 1, unroll=2)
        def _relu(r):
            for j in range(0, _HH, _LANE):
                hv = hbuf[b, r, pl.ds(j, _LANE)]
                ev = ebuf[b, r, pl.ds(j, _LANE)]
                mbuf[b, r, pl.ds(j, _LANE)] = jnp.maximum(hv + ev, 0.0)

    def issue_scatter(b):
        pltpu.async_copy(mbuf.at[b], agg_sh.at[dstb.at[b]], ssem[b], add=True)

    def wait_scatter(b):
        pltpu.make_async_copy(mbuf.at[b], agg_sh.at[dstb.at[b]],
                              ssem[b]).wait()

    @pl.loop(0, _TCH_E)
    def _chunk(t):
        gc = s + 16 * t
        pltpu.sync_copy(ip_hbm.at[pl.ds(gc * 4 * _CH, 4 * _CH)], ipack.at[0])
        pltpu.sync_copy(dp_hbm.at[pl.ds(gc * _CH, _CH)], dstb.at[0])
        compute_code(0)
        pltpu.sync_copy(h_hbm.at[ipack.at[0, pl.ds(0, _CH)]], hbuf.at[0])
        pltpu.sync_copy(etab_sh.at[codeb.at[0]], ebuf.at[0])
        relu_msg(0)
        pltpu.sync_copy(mbuf.at[0], agg_sh.at[dstb.at[0]], add=True)

    plsc.subcore_barrier()

    nblk2 = 4 + jnp.where(s < 14, 1, 0)

    @pl.loop(0, nblk2)
    def _dblk(k):
        rb = (s + 16 * k) * _CH
        pltpu.sync_copy(agg_sh.at[pl.ds(rb, _CH)],
                        out_hbm.at[pl.ds(c * _N + rb, _CH)])

    @pl.when(s == 0)
    def _dtail():
        pltpu.sync_copy(agg_sh.at[pl.ds(78 * _CH, 16)],
                        out_hbm.at[pl.ds(c * _N + 78 * _CH, 16)])


def _tc_dense_layer(h, p, w1a, w1b, b1, w2, b2, gamma, beta):
    def body(h_ref, p_ref, w1a_ref, w1b_ref, b1_ref, w2_ref, b2_ref,
             g_ref, be_ref, o_ref):
        a_lo = h_ref[0:_N] + p_ref[0:_N]
        a_hi = h_ref[_N:2 * _N] + p_ref[_N:2 * _N]
        t = (jnp.dot(a_lo, w1a_ref[...], preferred_element_type=jnp.float32)
             + jnp.dot(a_hi, w1b_ref[...], preferred_element_type=jnp.float32))
        t = jnp.maximum(t + b1_ref[...], 0.0)
        u = jnp.dot(t, w2_ref[...], preferred_element_type=jnp.float32)
        u = u + b2_ref[...]
        mu = jnp.mean(u, axis=0, keepdims=True)
        var = jnp.mean((u - mu) * (u - mu), axis=0, keepdims=True)
        v = (u - mu) * lax.rsqrt(var + 1e-5) * g_ref[...] + be_ref[...]
        v = jnp.maximum(v, 0.0)
        o_ref[0:_N] = h_ref[0:_N] + v[:, 0:_HH]
        o_ref[_N:2 * _N] = h_ref[_N:2 * _N] + v[:, _HH:_H]

    return pl.pallas_call(
        body,
        out_shape=jax.ShapeDtypeStruct((2 * _N, _HH), jnp.float32),
    )(h, p, w1a, w1b, b1, w2, b2, gamma, beta)


def _tc_pool_mlp(h, batchT, w_lo, w_hi, mlp_b):
    def body(h_ref, b_ref, wlo_ref, whi_ref, bias_ref, o_ref):
        gid = lax.broadcasted_iota(jnp.int32, (_G, _N), 0)
        oh = (b_ref[...] == gid).astype(jnp.float32)
        sums_lo = jnp.dot(oh, h_ref[0:_N], preferred_element_type=jnp.float32)
        sums_hi = jnp.dot(oh, h_ref[_N:2 * _N],
                          preferred_element_type=jnp.float32)
        inv = 1.0 / jnp.maximum(jnp.sum(oh, axis=1), 1.0)
        o_ref[...] = (jnp.dot(sums_lo * inv[:, None], wlo_ref[...],
                              preferred_element_type=jnp.float32)
                      + jnp.dot(sums_hi * inv[:, None], whi_ref[...],
                                preferred_element_type=jnp.float32)
                      + bias_ref[...])

    return pl.pallas_call(
        body,
        out_shape=jax.ShapeDtypeStruct((_G, _H), jnp.float32),
    )(h, batchT, w_lo, w_hi, mlp_b)


def kernel(x, edge_index, edge_attr, batch, atom_emb, bond_emb,
           W1, b1, W2, b2, bn_gamma, bn_beta, mlp_W, mlp_b):
    xpad = jnp.pad(x.astype(jnp.int32), ((0, _NPAD - _N), (0, 0)))
    xP = xpad.reshape(_NPAD // _CH, _CH, 9).transpose(0, 2, 1).reshape(-1)
    tab_full = atom_emb.reshape(9 * 128, _H)
    tab_cat = jnp.concatenate([tab_full[:, :_HH], tab_full[:, _HH:]], axis=0)
    src = jnp.pad(edge_index[0].astype(jnp.int32), (0, _EPAD - _E))
    dstp = jnp.pad(edge_index[1].astype(jnp.int32), (0, _EPAD - _E),
                   constant_values=_N)
    ea = jnp.pad(edge_attr.astype(jnp.int32), ((0, _EPAD - _E), (0, 0)))
    ipack = jnp.stack([src.reshape(_NCHUNKS_E, _CH),
                       ea[:, 0].reshape(_NCHUNKS_E, _CH),
                       ea[:, 1].reshape(_NCHUNKS_E, _CH),
                       ea[:, 2].reshape(_NCHUNKS_E, _CH)],
                      axis=1).reshape(-1)
    be_cat = jnp.concatenate([bond_emb[:, :, :_HH].reshape(-1),
                              bond_emb[:, :, _HH:].reshape(-1)])
    batchT = jnp.broadcast_to(batch.astype(jnp.int32)[None, :], (_G, _N))

    h0 = _sc_atom_encoder(xP, tab_cat)
    h = jnp.concatenate([h0[:_N], h0[_NPAD:_NPAD + _N]], axis=0)
    for i in range(3):
        p = _sc_edge_stage(h, ipack, dstp, be_cat)
        h = _tc_dense_layer(h, p, W1[i][:_HH], W1[i][_HH:],
                            b1[i].reshape(1, _H),
                            W2[i], b2[i].reshape(1, _H),
                            bn_gamma[i].reshape(1, _H),
                            bn_beta[i].reshape(1, _H))
    return _tc_pool_mlp(h, batchT, mlp_W[:_HH], mlp_W[_HH:],
                        mlp_b.reshape(1, _H))


# speedup vs baseline: 1.4027x; 1.4027x over previous
import functools

import jax
import jax.numpy as jnp
from jax import lax
from jax.experimental import pallas as pl
from jax.experimental.pallas import tpu as pltpu
from jax.experimental.pallas import tpu_sc as plsc

_N = 10000
_NPAD = 10240
_E = 320000
_EPAD = 327680
_NAGG = 10016
_H = 128
_HH = 64
_G = 64
_CH = 128
_LANE = 16
_NCHUNKS_E = _EPAD // _CH
_TCH_E = _NCHUNKS_E // 16

_mesh = plsc.VectorSubcoreMesh(core_axis_name="c", subcore_axis_name="s")
_sc_params = pltpu.CompilerParams(use_tc_tiling_on_sc=False)


@functools.partial(pl.kernel,
                   out_type=jax.ShapeDtypeStruct((2 * _NPAD, _HH), jnp.float32),
                   mesh=_mesh,
                   compiler_params=_sc_params,
                   scratch_types=[
                       pltpu.VMEM((9 * _CH,), jnp.int32),
                       pltpu.VMEM((2, _CH, _HH), jnp.float32),
                       pltpu.VMEM((_CH, _HH), jnp.float32),
                       pltpu.SemaphoreType.DMA,
                       pltpu.SemaphoreType.DMA,
                   ])
def _sc_atom_encoder(xP_hbm, tab_hbm, h0_hbm, ibuf, rows, acc, g0, g1):
    c = lax.axis_index("c")
    s = lax.axis_index("s")
    sems = (g0, g1)

    @pl.loop(0, _NPAD // _CH // 16)
    def _chunk(t):
        gc = s + 16 * t
        pltpu.sync_copy(xP_hbm.at[pl.ds(gc * 9 * _CH, 9 * _CH)], ibuf)

        @plsc.parallel_loop(0, 9 * _CH, _LANE)
        def _off(i):
            ibuf[pl.ds(i, _LANE)] = (ibuf[pl.ds(i, _LANE)]
                                     + c * (9 * _H) + (i // _CH) * _H)

        pltpu.sync_copy(tab_hbm.at[ibuf.at[pl.ds(0, _CH)]], rows.at[0])
        for f in range(1, 9):
            b = f % 2
            pltpu.sync_copy(tab_hbm.at[ibuf.at[pl.ds(f * _CH, _CH)]],
                            rows.at[b])
            pb = (f - 1) % 2
            if f == 1:
                @plsc.parallel_loop(0, _CH)
                def _cp(r):
                    for j in range(0, _HH, _LANE):
                        acc[r, pl.ds(j, _LANE)] = rows[pb, r, pl.ds(j, _LANE)]
            else:
                @plsc.parallel_loop(0, _CH)
                def _add(r):
                    for j in range(0, _HH, _LANE):
                        acc[r, pl.ds(j, _LANE)] = (
                            acc[r, pl.ds(j, _LANE)]
                            + rows[pb, r, pl.ds(j, _LANE)])
        @plsc.parallel_loop(0, _CH)
        def _add8(r):
            for j in range(0, _HH, _LANE):
                acc[r, pl.ds(j, _LANE)] = (
                    acc[r, pl.ds(j, _LANE)] + rows[0, r, pl.ds(j, _LANE)])

        pltpu.sync_copy(acc, h0_hbm.at[pl.ds(c * _NPAD + gc * _CH, _CH)])


@functools.partial(pl.kernel,
                   out_type=jax.ShapeDtypeStruct((2 * _N, _HH), jnp.float32),
                   mesh=_mesh,
                   compiler_params=_sc_params,
                   scratch_types=[
                       pltpu.VMEM((4, 4 * _CH), jnp.int32),
                       pltpu.VMEM((4, _CH), jnp.int32),
                       pltpu.VMEM((4, _CH), jnp.int32),
                       pltpu.VMEM((3 * 8 * _HH,), jnp.float32),
                       pltpu.VMEM((4, _CH, _HH), jnp.float32),
                       pltpu.VMEM((4, _CH, _HH), jnp.float32),
                       pltpu.VMEM((2, _CH, _HH), jnp.float32),
                       pltpu.VMEM_SHARED((_NAGG, _HH), jnp.float32),
                       pltpu.VMEM_SHARED((512, _HH), jnp.float32),
                       [pltpu.SemaphoreType.DMA] * 4,
                       [pltpu.SemaphoreType.DMA] * 4,
                       [pltpu.SemaphoreType.DMA] * 4,
                   ])
def _sc_edge_stage(h_hbm, ip_hbm, dp_hbm, be_hbm, out_hbm,
                   ipack, dstb, codeb, bflat, hbuf, ebuf, mbuf,
                   agg_sh, etab_sh, isem, gsem, ssem):
    c = lax.axis_index("c")
    s = lax.axis_index("s")

    pltpu.sync_copy(be_hbm.at[pl.ds(c * (24 * _HH), 24 * _HH)], bflat)
    for rl in range(32):
        r = s * 32 + rl
        c0 = r // 64
        c1 = (r // 8) % 8
        c2 = r % 8
        for j in range(0, _HH, _LANE):
            ebuf[0, rl, pl.ds(j, _LANE)] = (
                bflat[pl.ds(c0 * _HH + j, _LANE)]
                + bflat[pl.ds(8 * _HH + c1 * _HH + j, _LANE)]
                + bflat[pl.ds(16 * _HH + c2 * _HH + j, _LANE)])
    pltpu.sync_copy(ebuf.at[0, pl.ds(0, 32)], etab_sh.at[pl.ds(s * 32, 32)])

    @plsc.parallel_loop(0, _CH)
    def _zrow(r):
        for j in range(0, _HH, _LANE):
            mbuf[0, r, pl.ds(j, _LANE)] = jnp.zeros((_LANE,), jnp.float32)

    nblk = 4 + jnp.where(s < 14, 1, 0)

    @pl.loop(0, nblk)
    def _zblk(k):
        pltpu.sync_copy(mbuf.at[0], agg_sh.at[pl.ds((s + 16 * k) * _CH, _CH)])

    @pl.when(s == 0)
    def _ztail():
        pltpu.sync_copy(mbuf.at[0, pl.ds(0, 32)],
                        agg_sh.at[pl.ds(78 * _CH, 32)])

    plsc.subcore_barrier()

    def issue_idx(t, b):
        gc = s + 16 * t
        pltpu.async_copy(ip_hbm.at[pl.ds(gc * 4 * _CH, 4 * _CH)],
                         ipack.at[b], isem[b])
        pltpu.async_copy(dp_hbm.at[pl.ds(gc * _CH, _CH)], dstb.at[b], isem[b])

    def wait_idx(t, b):
        gc = s + 16 * t
        pltpu.make_async_copy(ip_hbm.at[pl.ds(gc * 4 * _CH, 4 * _CH)],
                              ipack.at[b], isem[b]).wait()
        pltpu.make_async_copy(dp_hbm.at[pl.ds(gc * _CH, _CH)], dstb.at[b],
                              isem[b]).wait()

    def compute_code(b):
        @plsc.parallel_loop(0, _CH, _LANE)
        def _srcoff(i):
            ipack[b, pl.ds(i, _LANE)] = ipack[b, pl.ds(i, _LANE)] + c * _N

        @plsc.parallel_loop(0, _CH, _LANE)
        def _code(i):
            codeb[b, pl.ds(i, _LANE)] = (
                ipack[b, pl.ds(_CH + i, _LANE)] * 64
                + ipack[b, pl.ds(2 * _CH + i, _LANE)] * 8
                + ipack[b, pl.ds(3 * _CH + i, _LANE)])

    def relu_msg(b, mb):
        @plsc.parallel_loop(0, _CH, 1, unroll=2)
        def _relu(r):
            for j in range(0, _HH, _LANE):
                hv = hbuf[b, r, pl.ds(j, _LANE)]
                ev = ebuf[b, r, pl.ds(j, _LANE)]
                mbuf[mb, r, pl.ds(j, _LANE)] = jnp.maximum(hv + ev, 0.0)

    @pl.loop(0, _TCH_E // 4)
    def _quad(qq):
        idma = []
        for kk in range(4):
            gc = s + 16 * (4 * qq + kk)
            idma.append(pltpu.async_copy(
                ip_hbm.at[pl.ds(gc * 4 * _CH, 4 * _CH)], ipack.at[kk],
                isem[kk]))
            idma.append(pltpu.async_copy(
                dp_hbm.at[pl.ds(gc * _CH, _CH)], dstb.at[kk], isem[kk]))
        for d in idma:
            d.wait()
        gdma = []
        for kk in range(4):
            compute_code(kk)
            gdma.append(pltpu.async_copy(
                h_hbm.at[ipack.at[kk, pl.ds(0, _CH)]], hbuf.at[kk],
                gsem[kk]))
        for kk in range(4):
            pltpu.sync_copy(etab_sh.at[codeb.at[kk]], ebuf.at[kk])
        for kk in range(4):
            gdma[kk].wait()
        for kk in range(4):
            relu_msg(kk, kk % 2)
            pltpu.sync_copy(mbuf.at[kk % 2], agg_sh.at[dstb.at[kk]],
                            add=True)

    plsc.subcore_barrier()

    nblk2 = 4 + jnp.where(s < 14, 1, 0)

    @pl.loop(0, nblk2)
    def _dblk(k):
        rb = (s + 16 * k) * _CH
        pltpu.sync_copy(agg_sh.at[pl.ds(rb, _CH)],
                        out_hbm.at[pl.ds(c * _N + rb, _CH)])

    @pl.when(s == 0)
    def _dtail():
        pltpu.sync_copy(agg_sh.at[pl.ds(78 * _CH, 16)],
                        out_hbm.at[pl.ds(c * _N + 78 * _CH, 16)])


def _tc_dense_layer(h, p, w1a, w1b, b1, w2, b2, gamma, beta):
    def body(h_ref, p_ref, w1a_ref, w1b_ref, b1_ref, w2_ref, b2_ref,
             g_ref, be_ref, o_ref):
        a_lo = h_ref[0:_N] + p_ref[0:_N]
        a_hi = h_ref[_N:2 * _N] + p_ref[_N:2 * _N]
        t = (jnp.dot(a_lo, w1a_ref[...], preferred_element_type=jnp.float32)
             + jnp.dot(a_hi, w1b_ref[...], preferred_element_type=jnp.float32))
        t = jnp.maximum(t + b1_ref[...], 0.0)
        u = jnp.dot(t, w2_ref[...], preferred_element_type=jnp.float32)
        u = u + b2_ref[...]
        mu = jnp.mean(u, axis=0, keepdims=True)
        var = jnp.mean((u - mu) * (u - mu), axis=0, keepdims=True)
        v = (u - mu) * lax.rsqrt(var + 1e-5) * g_ref[...] + be_ref[...]
        v = jnp.maximum(v, 0.0)
        o_ref[0:_N] = h_ref[0:_N] + v[:, 0:_HH]
        o_ref[_N:2 * _N] = h_ref[_N:2 * _N] + v[:, _HH:_H]

    return pl.pallas_call(
        body,
        out_shape=jax.ShapeDtypeStruct((2 * _N, _HH), jnp.float32),
    )(h, p, w1a, w1b, b1, w2, b2, gamma, beta)


def _tc_pool_mlp(h, batchT, w_lo, w_hi, mlp_b):
    def body(h_ref, b_ref, wlo_ref, whi_ref, bias_ref, o_ref):
        gid = lax.broadcasted_iota(jnp.int32, (_G, _N), 0)
        oh = (b_ref[...] == gid).astype(jnp.float32)
        sums_lo = jnp.dot(oh, h_ref[0:_N], preferred_element_type=jnp.float32)
        sums_hi = jnp.dot(oh, h_ref[_N:2 * _N],
                          preferred_element_type=jnp.float32)
        inv = 1.0 / jnp.maximum(jnp.sum(oh, axis=1), 1.0)
        o_ref[...] = (jnp.dot(sums_lo * inv[:, None], wlo_ref[...],
                              preferred_element_type=jnp.float32)
                      + jnp.dot(sums_hi * inv[:, None], whi_ref[...],
                                preferred_element_type=jnp.float32)
                      + bias_ref[...])

    return pl.pallas_call(
        body,
        out_shape=jax.ShapeDtypeStruct((_G, _H), jnp.float32),
    )(h, batchT, w_lo, w_hi, mlp_b)


def kernel(x, edge_index, edge_attr, batch, atom_emb, bond_emb,
           W1, b1, W2, b2, bn_gamma, bn_beta, mlp_W, mlp_b):
    xpad = jnp.pad(x.astype(jnp.int32), ((0, _NPAD - _N), (0, 0)))
    xP = xpad.reshape(_NPAD // _CH, _CH, 9).transpose(0, 2, 1).reshape(-1)
    tab_full = atom_emb.reshape(9 * 128, _H)
    tab_cat = jnp.concatenate([tab_full[:, :_HH], tab_full[:, _HH:]], axis=0)
    src = jnp.pad(edge_index[0].astype(jnp.int32), (0, _EPAD - _E))
    dstp = jnp.pad(edge_index[1].astype(jnp.int32), (0, _EPAD - _E),
                   constant_values=_N)
    ea = jnp.pad(edge_attr.astype(jnp.int32), ((0, _EPAD - _E), (0, 0)))
    ipack = jnp.stack([src.reshape(_NCHUNKS_E, _CH),
                       ea[:, 0].reshape(_NCHUNKS_E, _CH),
                       ea[:, 1].reshape(_NCHUNKS_E, _CH),
                       ea[:, 2].reshape(_NCHUNKS_E, _CH)],
                      axis=1).reshape(-1)
    be_cat = jnp.concatenate([bond_emb[:, :, :_HH].reshape(-1),
                              bond_emb[:, :, _HH:].reshape(-1)])
    batchT = jnp.broadcast_to(batch.astype(jnp.int32)[None, :], (_G, _N))

    h0 = _sc_atom_encoder(xP, tab_cat)
    h = jnp.concatenate([h0[:_N], h0[_NPAD:_NPAD + _N]], axis=0)
    for i in range(3):
        p = _sc_edge_stage(h, ipack, dstp, be_cat)
        h = _tc_dense_layer(h, p, W1[i][:_HH], W1[i][_HH:],
                            b1[i].reshape(1, _H),
                            W2[i], b2[i].reshape(1, _H),
                            bn_gamma[i].reshape(1, _H),
                            bn_beta[i].reshape(1, _H))
    return _tc_pool_mlp(h, batchT, mlp_W[:_HH], mlp_W[_HH:],
                        mlp_b.reshape(1, _H))


# speedup vs baseline: 1.5074x; 1.0747x over previous
import functools

import jax
import jax.numpy as jnp
from jax import lax
from jax.experimental import pallas as pl
from jax.experimental.pallas import tpu as pltpu
from jax.experimental.pallas import tpu_sc as plsc

_N = 10000
_NPAD = 10240
_E = 320000
_EPAD = 327680
_NAGG = 10016
_H = 128
_HH = 64
_G = 64
_CH = 128
_LANE = 16
_NCHUNKS_E = _EPAD // _CH
_TCH_E = _NCHUNKS_E // 16

_mesh = plsc.VectorSubcoreMesh(core_axis_name="c", subcore_axis_name="s")
_sc_params = pltpu.CompilerParams(use_tc_tiling_on_sc=False)


@functools.partial(pl.kernel,
                   out_type=jax.ShapeDtypeStruct((2 * _NPAD, _HH), jnp.float32),
                   mesh=_mesh,
                   compiler_params=_sc_params,
                   scratch_types=[
                       pltpu.VMEM((9 * _CH,), jnp.int32),
                       pltpu.VMEM((2, _CH, _HH), jnp.float32),
                       pltpu.VMEM((_CH, _HH), jnp.float32),
                       pltpu.SemaphoreType.DMA,
                       pltpu.SemaphoreType.DMA,
                   ])
def _sc_atom_encoder(xP_hbm, tab_hbm, h0_hbm, ibuf, rows, acc, g0, g1):
    c = lax.axis_index("c")
    s = lax.axis_index("s")
    sems = (g0, g1)

    @pl.loop(0, _NPAD // _CH // 16)
    def _chunk(t):
        gc = s + 16 * t
        pltpu.sync_copy(xP_hbm.at[pl.ds(gc * 9 * _CH, 9 * _CH)], ibuf)

        @plsc.parallel_loop(0, 9 * _CH, _LANE)
        def _off(i):
            ibuf[pl.ds(i, _LANE)] = (ibuf[pl.ds(i, _LANE)]
                                     + c * (9 * _H) + (i // _CH) * _H)

        pltpu.sync_copy(tab_hbm.at[ibuf.at[pl.ds(0, _CH)]], rows.at[0])
        for f in range(1, 9):
            b = f % 2
            pltpu.sync_copy(tab_hbm.at[ibuf.at[pl.ds(f * _CH, _CH)]],
                            rows.at[b])
            pb = (f - 1) % 2
            if f == 1:
                @plsc.parallel_loop(0, _CH)
                def _cp(r):
                    for j in range(0, _HH, _LANE):
                        acc[r, pl.ds(j, _LANE)] = rows[pb, r, pl.ds(j, _LANE)]
            else:
                @plsc.parallel_loop(0, _CH)
                def _add(r):
                    for j in range(0, _HH, _LANE):
                        acc[r, pl.ds(j, _LANE)] = (
                            acc[r, pl.ds(j, _LANE)]
                            + rows[pb, r, pl.ds(j, _LANE)])
        @plsc.parallel_loop(0, _CH)
        def _add8(r):
            for j in range(0, _HH, _LANE):
                acc[r, pl.ds(j, _LANE)] = (
                    acc[r, pl.ds(j, _LANE)] + rows[0, r, pl.ds(j, _LANE)])

        pltpu.sync_copy(acc, h0_hbm.at[pl.ds(c * _NPAD + gc * _CH, _CH)])


@functools.partial(pl.kernel,
                   out_type=jax.ShapeDtypeStruct((2 * _N, _HH), jnp.float32),
                   mesh=_mesh,
                   compiler_params=_sc_params,
                   scratch_types=[
                       pltpu.VMEM((4, 4 * _CH), jnp.int32),
                       pltpu.VMEM((4, _CH), jnp.int32),
                       pltpu.VMEM((4, _CH), jnp.int32),
                       pltpu.VMEM((3 * 8 * _HH,), jnp.float32),
                       pltpu.VMEM((4, _CH, _HH), jnp.float32),
                       pltpu.VMEM((4, _CH, _HH), jnp.float32),
                       pltpu.VMEM((2, _CH, _HH), jnp.float32),
                       pltpu.VMEM_SHARED((_NAGG, _HH), jnp.float32),
                       pltpu.VMEM_SHARED((512, _HH), jnp.float32),
                       [pltpu.SemaphoreType.DMA] * 4,
                       [pltpu.SemaphoreType.DMA] * 4,
                       [pltpu.SemaphoreType.DMA] * 4,
                   ])
def _sc_edge_stage(h_hbm, ip_hbm, dp_hbm, be_hbm, out_hbm,
                   ipack, dstb, codeb, bflat, hbuf, ebuf, mbuf,
                   agg_sh, etab_sh, isem, gsem, ssem):
    c = lax.axis_index("c")
    s = lax.axis_index("s")

    pltpu.sync_copy(be_hbm.at[pl.ds(c * (24 * _HH), 24 * _HH)], bflat)
    for rl in range(32):
        r = s * 32 + rl
        c0 = r // 64
        c1 = (r // 8) % 8
        c2 = r % 8
        for j in range(0, _HH, _LANE):
            ebuf[0, rl, pl.ds(j, _LANE)] = (
                bflat[pl.ds(c0 * _HH + j, _LANE)]
                + bflat[pl.ds(8 * _HH + c1 * _HH + j, _LANE)]
                + bflat[pl.ds(16 * _HH + c2 * _HH + j, _LANE)])
    pltpu.sync_copy(ebuf.at[0, pl.ds(0, 32)], etab_sh.at[pl.ds(s * 32, 32)])

    @plsc.parallel_loop(0, _CH)
    def _zrow(r):
        for j in range(0, _HH, _LANE):
            mbuf[0, r, pl.ds(j, _LANE)] = jnp.zeros((_LANE,), jnp.float32)

    nblk = 4 + jnp.where(s < 14, 1, 0)

    @pl.loop(0, nblk)
    def _zblk(k):
        pltpu.sync_copy(mbuf.at[0], agg_sh.at[pl.ds((s + 16 * k) * _CH, _CH)])

    @pl.when(s == 0)
    def _ztail():
        pltpu.sync_copy(mbuf.at[0, pl.ds(0, 32)],
                        agg_sh.at[pl.ds(78 * _CH, 32)])

    plsc.subcore_barrier()

    def issue_idx(t, b):
        gc = s + 16 * t
        pltpu.async_copy(ip_hbm.at[pl.ds(gc * 4 * _CH, 4 * _CH)],
                         ipack.at[b], isem[b])
        pltpu.async_copy(dp_hbm.at[pl.ds(gc * _CH, _CH)], dstb.at[b], isem[b])

    def wait_idx(t, b):
        gc = s + 16 * t
        pltpu.make_async_copy(ip_hbm.at[pl.ds(gc * 4 * _CH, 4 * _CH)],
                              ipack.at[b], isem[b]).wait()
        pltpu.make_async_copy(dp_hbm.at[pl.ds(gc * _CH, _CH)], dstb.at[b],
                              isem[b]).wait()

    def compute_code(b):
        @plsc.parallel_loop(0, _CH, _LANE)
        def _srcoff(i):
            ipack[b, pl.ds(i, _LANE)] = ipack[b, pl.ds(i, _LANE)] + c * _N

        @plsc.parallel_loop(0, _CH, _LANE)
        def _code(i):
            codeb[b, pl.ds(i, _LANE)] = (
                ipack[b, pl.ds(_CH + i, _LANE)] * 64
                + ipack[b, pl.ds(2 * _CH + i, _LANE)] * 8
                + ipack[b, pl.ds(3 * _CH + i, _LANE)])

    def relu_msg(b, mb):
        @plsc.parallel_loop(0, _CH, 1, unroll=2)
        def _relu(r):
            for j in range(0, _HH, _LANE):
                hv = hbuf[b, r, pl.ds(j, _LANE)]
                ev = ebuf[b, r, pl.ds(j, _LANE)]
                mbuf[mb, r, pl.ds(j, _LANE)] = jnp.maximum(hv + ev, 0.0)

    @pl.loop(0, _TCH_E // 4)
    def _quad(qq):
        idma = []
        for kk in range(4):
            gc = s + 16 * (4 * qq + kk)
            idma.append(pltpu.async_copy(
                ip_hbm.at[pl.ds(gc * 4 * _CH, 4 * _CH)], ipack.at[kk],
                isem[kk]))
            idma.append(pltpu.async_copy(
                dp_hbm.at[pl.ds(gc * _CH, _CH)], dstb.at[kk], isem[kk]))
        for d in idma:
            d.wait()
        gdma = []
        for kk in range(4):
            compute_code(kk)
            gdma.append(pltpu.async_copy(
                h_hbm.at[ipack.at[kk, pl.ds(0, _CH)]], hbuf.at[kk],
                gsem[kk]))
        for kk in range(4):
            gdma.append(pltpu.async_copy(
                etab_sh.at[codeb.at[kk]], ebuf.at[kk], isem[kk]))
        for d in gdma:
            d.wait()
        sdma = []
        for kk in range(4):
            if kk >= 2:
                sdma[kk - 2].wait()
            relu_msg(kk, kk % 2)
            sdma.append(pltpu.async_copy(
                mbuf.at[kk % 2], agg_sh.at[dstb.at[kk]], ssem[kk], add=True))
        sdma[2].wait()
        sdma[3].wait()

    plsc.subcore_barrier()

    nblk2 = 4 + jnp.where(s < 14, 1, 0)

    @pl.loop(0, nblk2)
    def _dblk(k):
        rb = (s + 16 * k) * _CH
        pltpu.sync_copy(agg_sh.at[pl.ds(rb, _CH)],
                        out_hbm.at[pl.ds(c * _N + rb, _CH)])

    @pl.when(s == 0)
    def _dtail():
        pltpu.sync_copy(agg_sh.at[pl.ds(78 * _CH, 16)],
                        out_hbm.at[pl.ds(c * _N + 78 * _CH, 16)])


def _tc_dense_layer(h, p, w1a, w1b, b1, w2, b2, gamma, beta):
    def body(h_ref, p_ref, w1a_ref, w1b_ref, b1_ref, w2_ref, b2_ref,
             g_ref, be_ref, o_ref):
        a_lo = h_ref[0:_N] + p_ref[0:_N]
        a_hi = h_ref[_N:2 * _N] + p_ref[_N:2 * _N]
        t = (jnp.dot(a_lo, w1a_ref[...], preferred_element_type=jnp.float32)
             + jnp.dot(a_hi, w1b_ref[...], preferred_element_type=jnp.float32))
        t = jnp.maximum(t + b1_ref[...], 0.0)
        u = jnp.dot(t, w2_ref[...], preferred_element_type=jnp.float32)
        u = u + b2_ref[...]
        mu = jnp.mean(u, axis=0, keepdims=True)
        var = jnp.mean((u - mu) * (u - mu), axis=0, keepdims=True)
        v = (u - mu) * lax.rsqrt(var + 1e-5) * g_ref[...] + be_ref[...]
        v = jnp.maximum(v, 0.0)
        o_ref[0:_N] = h_ref[0:_N] + v[:, 0:_HH]
        o_ref[_N:2 * _N] = h_ref[_N:2 * _N] + v[:, _HH:_H]

    return pl.pallas_call(
        body,
        out_shape=jax.ShapeDtypeStruct((2 * _N, _HH), jnp.float32),
    )(h, p, w1a, w1b, b1, w2, b2, gamma, beta)


def _tc_pool_mlp(h, batchT, w_lo, w_hi, mlp_b):
    def body(h_ref, b_ref, wlo_ref, whi_ref, bias_ref, o_ref):
        gid = lax.broadcasted_iota(jnp.int32, (_G, _N), 0)
        oh = (b_ref[...] == gid).astype(jnp.float32)
        sums_lo = jnp.dot(oh, h_ref[0:_N], preferred_element_type=jnp.float32)
        sums_hi = jnp.dot(oh, h_ref[_N:2 * _N],
                          preferred_element_type=jnp.float32)
        inv = 1.0 / jnp.maximum(jnp.sum(oh, axis=1), 1.0)
        o_ref[...] = (jnp.dot(sums_lo * inv[:, None], wlo_ref[...],
                              preferred_element_type=jnp.float32)
                      + jnp.dot(sums_hi * inv[:, None], whi_ref[...],
                                preferred_element_type=jnp.float32)
                      + bias_ref[...])

    return pl.pallas_call(
        body,
        out_shape=jax.ShapeDtypeStruct((_G, _H), jnp.float32),
    )(h, batchT, w_lo, w_hi, mlp_b)


def kernel(x, edge_index, edge_attr, batch, atom_emb, bond_emb,
           W1, b1, W2, b2, bn_gamma, bn_beta, mlp_W, mlp_b):
    xpad = jnp.pad(x.astype(jnp.int32), ((0, _NPAD - _N), (0, 0)))
    xP = xpad.reshape(_NPAD // _CH, _CH, 9).transpose(0, 2, 1).reshape(-1)
    tab_full = atom_emb.reshape(9 * 128, _H)
    tab_cat = jnp.concatenate([tab_full[:, :_HH], tab_full[:, _HH:]], axis=0)
    src = jnp.pad(edge_index[0].astype(jnp.int32), (0, _EPAD - _E))
    dstp = jnp.pad(edge_index[1].astype(jnp.int32), (0, _EPAD - _E),
                   constant_values=_N)
    ea = jnp.pad(edge_attr.astype(jnp.int32), ((0, _EPAD - _E), (0, 0)))
    ipack = jnp.stack([src.reshape(_NCHUNKS_E, _CH),
                       ea[:, 0].reshape(_NCHUNKS_E, _CH),
                       ea[:, 1].reshape(_NCHUNKS_E, _CH),
                       ea[:, 2].reshape(_NCHUNKS_E, _CH)],
                      axis=1).reshape(-1)
    be_cat = jnp.concatenate([bond_emb[:, :, :_HH].reshape(-1),
                              bond_emb[:, :, _HH:].reshape(-1)])
    batchT = jnp.broadcast_to(batch.astype(jnp.int32)[None, :], (_G, _N))

    h0 = _sc_atom_encoder(xP, tab_cat)
    h = jnp.concatenate([h0[:_N], h0[_NPAD:_NPAD + _N]], axis=0)
    for i in range(3):
        p = _sc_edge_stage(h, ipack, dstp, be_cat)
        h = _tc_dense_layer(h, p, W1[i][:_HH], W1[i][_HH:],
                            b1[i].reshape(1, _H),
                            W2[i], b2[i].reshape(1, _H),
                            bn_gamma[i].reshape(1, _H),
                            bn_beta[i].reshape(1, _H))
    return _tc_pool_mlp(h, batchT, mlp_W[:_HH], mlp_W[_HH:],
                        mlp_b.reshape(1, _H))


# speedup vs baseline: 1.5587x; 1.0340x over previous
import functools

import jax
import jax.numpy as jnp
from jax import lax
from jax.experimental import pallas as pl
from jax.experimental.pallas import tpu as pltpu
from jax.experimental.pallas import tpu_sc as plsc

_N = 10000
_NPAD = 10240
_E = 320000
_EPAD = 327680
_NAGG = 10016
_H = 128
_HH = 64
_G = 64
_CH = 128
_LANE = 16
_NCHUNKS_E = _EPAD // _CH
_TCH_E = _NCHUNKS_E // 16

_mesh = plsc.VectorSubcoreMesh(core_axis_name="c", subcore_axis_name="s")
_sc_params = pltpu.CompilerParams(use_tc_tiling_on_sc=False)


@functools.partial(pl.kernel,
                   out_type=jax.ShapeDtypeStruct((2 * _NPAD, _HH), jnp.float32),
                   mesh=_mesh,
                   compiler_params=_sc_params,
                   scratch_types=[
                       pltpu.VMEM((9 * _CH,), jnp.int32),
                       pltpu.VMEM((9, _CH, _HH), jnp.float32),
                       pltpu.VMEM((_CH, _HH), jnp.float32),
                       [pltpu.SemaphoreType.DMA] * 9,
                   ])
def _sc_atom_encoder(xP_hbm, tab_hbm, h0_hbm, ibuf, rows, acc, gsem):
    c = lax.axis_index("c")
    s = lax.axis_index("s")

    @pl.loop(0, _NPAD // _CH // 16)
    def _chunk(t):
        gc = s + 16 * t
        pltpu.sync_copy(xP_hbm.at[pl.ds(gc * 9 * _CH, 9 * _CH)], ibuf)

        @plsc.parallel_loop(0, 9 * _CH, _LANE)
        def _off(i):
            ibuf[pl.ds(i, _LANE)] = (ibuf[pl.ds(i, _LANE)]
                                     + c * (9 * _H) + (i // _CH) * _H)

        gdma = [pltpu.async_copy(tab_hbm.at[ibuf.at[pl.ds(f * _CH, _CH)]],
                                 rows.at[f], gsem[f]) for f in range(9)]
        for d in gdma:
            d.wait()

        @plsc.parallel_loop(0, _CH)
        def _acc(r):
            for j in range(0, _HH, _LANE):
                v = rows[0, r, pl.ds(j, _LANE)]
                for f in range(1, 9):
                    v = v + rows[f, r, pl.ds(j, _LANE)]
                acc[r, pl.ds(j, _LANE)] = v

        pltpu.sync_copy(acc, h0_hbm.at[pl.ds(c * _NPAD + gc * _CH, _CH)])


@functools.partial(pl.kernel,
                   out_type=jax.ShapeDtypeStruct((2 * _N, _HH), jnp.float32),
                   mesh=_mesh,
                   compiler_params=_sc_params,
                   scratch_types=[
                       pltpu.VMEM((4, 4 * _CH), jnp.int32),
                       pltpu.VMEM((4, _CH), jnp.int32),
                       pltpu.VMEM((4, _CH), jnp.int32),
                       pltpu.VMEM((3 * 8 * _HH,), jnp.float32),
                       pltpu.VMEM((4, _CH, _HH), jnp.float32),
                       pltpu.VMEM((4, _CH, _HH), jnp.float32),
                       pltpu.VMEM((2, _CH, _HH), jnp.float32),
                       pltpu.VMEM_SHARED((_NAGG, _HH), jnp.float32),
                       pltpu.VMEM_SHARED((512, _HH), jnp.float32),
                       [pltpu.SemaphoreType.DMA] * 4,
                       [pltpu.SemaphoreType.DMA] * 4,
                       [pltpu.SemaphoreType.DMA] * 4,
                   ])
def _sc_edge_stage(h_hbm, ip_hbm, dp_hbm, be_hbm, out_hbm,
                   ipack, dstb, codeb, bflat, hbuf, ebuf, mbuf,
                   agg_sh, etab_sh, isem, gsem, ssem):
    c = lax.axis_index("c")
    s = lax.axis_index("s")

    pltpu.sync_copy(be_hbm.at[pl.ds(c * (24 * _HH), 24 * _HH)], bflat)
    for rl in range(32):
        r = s * 32 + rl
        c0 = r // 64
        c1 = (r // 8) % 8
        c2 = r % 8
        for j in range(0, _HH, _LANE):
            ebuf[0, rl, pl.ds(j, _LANE)] = (
                bflat[pl.ds(c0 * _HH + j, _LANE)]
                + bflat[pl.ds(8 * _HH + c1 * _HH + j, _LANE)]
                + bflat[pl.ds(16 * _HH + c2 * _HH + j, _LANE)])
    pltpu.sync_copy(ebuf.at[0, pl.ds(0, 32)], etab_sh.at[pl.ds(s * 32, 32)])

    @plsc.parallel_loop(0, _CH)
    def _zrow(r):
        for j in range(0, _HH, _LANE):
            mbuf[0, r, pl.ds(j, _LANE)] = jnp.zeros((_LANE,), jnp.float32)

    nblk = 4 + jnp.where(s < 14, 1, 0)

    @pl.loop(0, nblk)
    def _zblk(k):
        pltpu.sync_copy(mbuf.at[0], agg_sh.at[pl.ds((s + 16 * k) * _CH, _CH)])

    @pl.when(s == 0)
    def _ztail():
        pltpu.sync_copy(mbuf.at[0, pl.ds(0, 32)],
                        agg_sh.at[pl.ds(78 * _CH, 32)])

    plsc.subcore_barrier()

    def issue_idx(t, b):
        gc = s + 16 * t
        pltpu.async_copy(ip_hbm.at[pl.ds(gc * 4 * _CH, 4 * _CH)],
                         ipack.at[b], isem[b])
        pltpu.async_copy(dp_hbm.at[pl.ds(gc * _CH, _CH)], dstb.at[b], isem[b])

    def wait_idx(t, b):
        gc = s + 16 * t
        pltpu.make_async_copy(ip_hbm.at[pl.ds(gc * 4 * _CH, 4 * _CH)],
                              ipack.at[b], isem[b]).wait()
        pltpu.make_async_copy(dp_hbm.at[pl.ds(gc * _CH, _CH)], dstb.at[b],
                              isem[b]).wait()

    def compute_code(b):
        @plsc.parallel_loop(0, _CH, _LANE)
        def _srcoff(i):
            ipack[b, pl.ds(i, _LANE)] = ipack[b, pl.ds(i, _LANE)] + c * _N

        @plsc.parallel_loop(0, _CH, _LANE)
        def _code(i):
            codeb[b, pl.ds(i, _LANE)] = (
                ipack[b, pl.ds(_CH + i, _LANE)] * 64
                + ipack[b, pl.ds(2 * _CH + i, _LANE)] * 8
                + ipack[b, pl.ds(3 * _CH + i, _LANE)])

    def relu_msg(b, mb):
        @plsc.parallel_loop(0, _CH, 1, unroll=2)
        def _relu(r):
            for j in range(0, _HH, _LANE):
                hv = hbuf[b, r, pl.ds(j, _LANE)]
                ev = ebuf[b, r, pl.ds(j, _LANE)]
                mbuf[mb, r, pl.ds(j, _LANE)] = jnp.maximum(hv + ev, 0.0)

    @pl.loop(0, _TCH_E // 4)
    def _quad(qq):
        idma = []
        for kk in range(4):
            gc = s + 16 * (4 * qq + kk)
            idma.append(pltpu.async_copy(
                ip_hbm.at[pl.ds(gc * 4 * _CH, 4 * _CH)], ipack.at[kk],
                isem[kk]))
            idma.append(pltpu.async_copy(
                dp_hbm.at[pl.ds(gc * _CH, _CH)], dstb.at[kk], isem[kk]))
        for d in idma:
            d.wait()
        gdma = []
        for kk in range(4):
            compute_code(kk)
            gdma.append(pltpu.async_copy(
                h_hbm.at[ipack.at[kk, pl.ds(0, _CH)]], hbuf.at[kk],
                gsem[kk]))
        for kk in range(4):
            gdma.append(pltpu.async_copy(
                etab_sh.at[codeb.at[kk]], ebuf.at[kk], isem[kk]))
        for d in gdma:
            d.wait()
        sdma = []
        for kk in range(4):
            if kk >= 2:
                sdma[kk - 2].wait()
            relu_msg(kk, kk % 2)
            sdma.append(pltpu.async_copy(
                mbuf.at[kk % 2], agg_sh.at[dstb.at[kk]], ssem[kk], add=True))
        sdma[2].wait()
        sdma[3].wait()

    plsc.subcore_barrier()

    nblk2 = 4 + jnp.where(s < 14, 1, 0)

    @pl.loop(0, nblk2)
    def _dblk(k):
        rb = (s + 16 * k) * _CH
        pltpu.sync_copy(agg_sh.at[pl.ds(rb, _CH)],
                        out_hbm.at[pl.ds(c * _N + rb, _CH)])

    @pl.when(s == 0)
    def _dtail():
        pltpu.sync_copy(agg_sh.at[pl.ds(78 * _CH, 16)],
                        out_hbm.at[pl.ds(c * _N + 78 * _CH, 16)])


def _tc_dense_layer(h, p, w1a, w1b, b1, w2, b2, gamma, beta):
    def body(h_ref, p_ref, w1a_ref, w1b_ref, b1_ref, w2_ref, b2_ref,
             g_ref, be_ref, o_ref):
        a_lo = h_ref[0:_N] + p_ref[0:_N]
        a_hi = h_ref[_N:2 * _N] + p_ref[_N:2 * _N]
        t = (jnp.dot(a_lo, w1a_ref[...], preferred_element_type=jnp.float32)
             + jnp.dot(a_hi, w1b_ref[...], preferred_element_type=jnp.float32))
        t = jnp.maximum(t + b1_ref[...], 0.0)
        u = jnp.dot(t, w2_ref[...], preferred_element_type=jnp.float32)
        u = u + b2_ref[...]
        mu = jnp.mean(u, axis=0, keepdims=True)
        var = jnp.mean((u - mu) * (u - mu), axis=0, keepdims=True)
        v = (u - mu) * lax.rsqrt(var + 1e-5) * g_ref[...] + be_ref[...]
        v = jnp.maximum(v, 0.0)
        o_ref[0:_N] = h_ref[0:_N] + v[:, 0:_HH]
        o_ref[_N:2 * _N] = h_ref[_N:2 * _N] + v[:, _HH:_H]

    return pl.pallas_call(
        body,
        out_shape=jax.ShapeDtypeStruct((2 * _N, _HH), jnp.float32),
    )(h, p, w1a, w1b, b1, w2, b2, gamma, beta)


def _tc_pool_mlp(h, batchT, w_lo, w_hi, mlp_b):
    def body(h_ref, b_ref, wlo_ref, whi_ref, bias_ref, o_ref):
        gid = lax.broadcasted_iota(jnp.int32, (_G, _N), 0)
        oh = (b_ref[...] == gid).astype(jnp.float32)
        sums_lo = jnp.dot(oh, h_ref[0:_N], preferred_element_type=jnp.float32)
        sums_hi = jnp.dot(oh, h_ref[_N:2 * _N],
                          preferred_element_type=jnp.float32)
        inv = 1.0 / jnp.maximum(jnp.sum(oh, axis=1), 1.0)
        o_ref[...] = (jnp.dot(sums_lo * inv[:, None], wlo_ref[...],
                              preferred_element_type=jnp.float32)
                      + jnp.dot(sums_hi * inv[:, None], whi_ref[...],
                                preferred_element_type=jnp.float32)
                      + bias_ref[...])

    return pl.pallas_call(
        body,
        out_shape=jax.ShapeDtypeStruct((_G, _H), jnp.float32),
    )(h, batchT, w_lo, w_hi, mlp_b)


def kernel(x, edge_index, edge_attr, batch, atom_emb, bond_emb,
           W1, b1, W2, b2, bn_gamma, bn_beta, mlp_W, mlp_b):
    xpad = jnp.pad(x.astype(jnp.int32), ((0, _NPAD - _N), (0, 0)))
    xP = xpad.reshape(_NPAD // _CH, _CH, 9).transpose(0, 2, 1).reshape(-1)
    tab_full = atom_emb.reshape(9 * 128, _H)
    tab_cat = jnp.concatenate([tab_full[:, :_HH], tab_full[:, _HH:]], axis=0)
    src = jnp.pad(edge_index[0].astype(jnp.int32), (0, _EPAD - _E))
    dstp = jnp.pad(edge_index[1].astype(jnp.int32), (0, _EPAD - _E),
                   constant_values=_N)
    ea = jnp.pad(edge_attr.astype(jnp.int32), ((0, _EPAD - _E), (0, 0)))
    ipack = jnp.stack([src.reshape(_NCHUNKS_E, _CH),
                       ea[:, 0].reshape(_NCHUNKS_E, _CH),
                       ea[:, 1].reshape(_NCHUNKS_E, _CH),
                       ea[:, 2].reshape(_NCHUNKS_E, _CH)],
                      axis=1).reshape(-1)
    be_cat = jnp.concatenate([bond_emb[:, :, :_HH].reshape(-1),
                              bond_emb[:, :, _HH:].reshape(-1)])
    batchT = jnp.broadcast_to(batch.astype(jnp.int32)[None, :], (_G, _N))

    h0 = _sc_atom_encoder(xP, tab_cat)
    h = jnp.concatenate([h0[:_N], h0[_NPAD:_NPAD + _N]], axis=0)
    for i in range(3):
        p = _sc_edge_stage(h, ipack, dstp, be_cat)
        h = _tc_dense_layer(h, p, W1[i][:_HH], W1[i][_HH:],
                            b1[i].reshape(1, _H),
                            W2[i], b2[i].reshape(1, _H),
                            bn_gamma[i].reshape(1, _H),
                            bn_beta[i].reshape(1, _H))
    return _tc_pool_mlp(h, batchT, mlp_W[:_HH], mlp_W[_HH:],
                        mlp_b.reshape(1, _H))


# speedup vs baseline: 1.5687x; 1.0064x over previous
import functools

import jax
import jax.numpy as jnp
from jax import lax
from jax.experimental import pallas as pl
from jax.experimental.pallas import tpu as pltpu
from jax.experimental.pallas import tpu_sc as plsc

_N = 10000
_NPAD = 10240
_E = 320000
_EPAD = 327680
_NAGG = 10016
_H = 128
_HH = 64
_G = 64
_CH = 128
_LANE = 16
_NCHUNKS_E = _EPAD // _CH
_TCH_E = _NCHUNKS_E // 16

_mesh = plsc.VectorSubcoreMesh(core_axis_name="c", subcore_axis_name="s")
_sc_params = pltpu.CompilerParams(use_tc_tiling_on_sc=False)


@functools.partial(pl.kernel,
                   out_type=jax.ShapeDtypeStruct((2 * _NPAD, _HH), jnp.float32),
                   mesh=_mesh,
                   compiler_params=_sc_params,
                   scratch_types=[
                       pltpu.VMEM((9 * _CH,), jnp.int32),
                       pltpu.VMEM((9, _CH, _HH), jnp.float32),
                       pltpu.VMEM((_CH, _HH), jnp.float32),
                       [pltpu.SemaphoreType.DMA] * 9,
                   ])
def _sc_atom_encoder(xP_hbm, tab_hbm, h0_hbm, ibuf, rows, acc, gsem):
    c = lax.axis_index("c")
    s = lax.axis_index("s")

    @pl.loop(0, _NPAD // _CH // 16)
    def _chunk(t):
        gc = s + 16 * t
        pltpu.sync_copy(xP_hbm.at[pl.ds(gc * 9 * _CH, 9 * _CH)], ibuf)

        @plsc.parallel_loop(0, 9 * _CH, _LANE)
        def _off(i):
            ibuf[pl.ds(i, _LANE)] = (ibuf[pl.ds(i, _LANE)]
                                     + c * (9 * _H) + (i // _CH) * _H)

        gdma = [pltpu.async_copy(tab_hbm.at[ibuf.at[pl.ds(f * _CH, _CH)]],
                                 rows.at[f], gsem[f]) for f in range(9)]
        for d in gdma:
            d.wait()

        @plsc.parallel_loop(0, _CH)
        def _acc(r):
            for j in range(0, _HH, _LANE):
                v = rows[0, r, pl.ds(j, _LANE)]
                for f in range(1, 9):
                    v = v + rows[f, r, pl.ds(j, _LANE)]
                acc[r, pl.ds(j, _LANE)] = v

        pltpu.sync_copy(acc, h0_hbm.at[pl.ds(c * _NPAD + gc * _CH, _CH)])


@functools.partial(pl.kernel,
                   out_type=jax.ShapeDtypeStruct((2 * _N, _HH), jnp.float32),
                   mesh=_mesh,
                   compiler_params=_sc_params,
                   scratch_types=[
                       pltpu.VMEM((4, 4 * _CH), jnp.int32),
                       pltpu.VMEM((4, _CH), jnp.int32),
                       pltpu.VMEM((4, _CH), jnp.int32),
                       pltpu.VMEM((3 * 8 * _HH,), jnp.float32),
                       pltpu.VMEM((4, _CH, _HH), jnp.float32),
                       pltpu.VMEM((4, _CH, _HH), jnp.float32),
                       pltpu.VMEM((2, _CH, _HH), jnp.float32),
                       pltpu.VMEM_SHARED((_NAGG, _HH), jnp.float32),
                       pltpu.VMEM_SHARED((512, _HH), jnp.float32),
                       [pltpu.SemaphoreType.DMA] * 4,
                       [pltpu.SemaphoreType.DMA] * 4,
                       [pltpu.SemaphoreType.DMA] * 4,
                   ])
def _sc_edge_stage(h_hbm, ip_hbm, dp_hbm, be_hbm, out_hbm,
                   ipack, dstb, codeb, bflat, hbuf, ebuf, mbuf,
                   agg_sh, etab_sh, isem, gsem, ssem):
    c = lax.axis_index("c")
    s = lax.axis_index("s")

    pltpu.sync_copy(be_hbm.at[pl.ds(c * (24 * _HH), 24 * _HH)], bflat)
    for rl in range(32):
        r = s * 32 + rl
        c0 = r // 64
        c1 = (r // 8) % 8
        c2 = r % 8
        for j in range(0, _HH, _LANE):
            ebuf[0, rl, pl.ds(j, _LANE)] = (
                bflat[pl.ds(c0 * _HH + j, _LANE)]
                + bflat[pl.ds(8 * _HH + c1 * _HH + j, _LANE)]
                + bflat[pl.ds(16 * _HH + c2 * _HH + j, _LANE)])
    pltpu.sync_copy(ebuf.at[0, pl.ds(0, 32)], etab_sh.at[pl.ds(s * 32, 32)])

    @plsc.parallel_loop(0, _CH)
    def _zrow(r):
        for j in range(0, _HH, _LANE):
            mbuf[0, r, pl.ds(j, _LANE)] = jnp.zeros((_LANE,), jnp.float32)

    nblk = 4 + jnp.where(s < 14, 1, 0)

    @pl.loop(0, nblk)
    def _zblk(k):
        pltpu.sync_copy(mbuf.at[0], agg_sh.at[pl.ds((s + 16 * k) * _CH, _CH)])

    @pl.when(s == 0)
    def _ztail():
        pltpu.sync_copy(mbuf.at[0, pl.ds(0, 32)],
                        agg_sh.at[pl.ds(78 * _CH, 32)])

    plsc.subcore_barrier()

    def issue_idx(t, b):
        gc = s + 16 * t
        pltpu.async_copy(ip_hbm.at[pl.ds(gc * 4 * _CH, 4 * _CH)],
                         ipack.at[b], isem[b])
        pltpu.async_copy(dp_hbm.at[pl.ds(gc * _CH, _CH)], dstb.at[b], isem[b])

    def wait_idx(t, b):
        gc = s + 16 * t
        pltpu.make_async_copy(ip_hbm.at[pl.ds(gc * 4 * _CH, 4 * _CH)],
                              ipack.at[b], isem[b]).wait()
        pltpu.make_async_copy(dp_hbm.at[pl.ds(gc * _CH, _CH)], dstb.at[b],
                              isem[b]).wait()

    def compute_code(b):
        @plsc.parallel_loop(0, _CH, _LANE)
        def _srcoff(i):
            ipack[b, pl.ds(i, _LANE)] = ipack[b, pl.ds(i, _LANE)] + c * _N

        @plsc.parallel_loop(0, _CH, _LANE)
        def _code(i):
            codeb[b, pl.ds(i, _LANE)] = (
                ipack[b, pl.ds(_CH + i, _LANE)] * 64
                + ipack[b, pl.ds(2 * _CH + i, _LANE)] * 8
                + ipack[b, pl.ds(3 * _CH + i, _LANE)])

    def relu_msg(b, mb):
        @plsc.parallel_loop(0, _CH, 1, unroll=2)
        def _relu(r):
            for j in range(0, _HH, _LANE):
                hv = hbuf[b, r, pl.ds(j, _LANE)]
                ev = ebuf[b, r, pl.ds(j, _LANE)]
                mbuf[mb, r, pl.ds(j, _LANE)] = jnp.maximum(hv + ev, 0.0)

    @pl.loop(0, _TCH_E // 4)
    def _quad(qq):
        idma = []
        for kk in range(4):
            gc = s + 16 * (4 * qq + kk)
            idma.append(pltpu.async_copy(
                ip_hbm.at[pl.ds(gc * 4 * _CH, 4 * _CH)], ipack.at[kk],
                isem[kk]))
            idma.append(pltpu.async_copy(
                dp_hbm.at[pl.ds(gc * _CH, _CH)], dstb.at[kk], isem[kk]))
        for d in idma:
            d.wait()
        gdma = []
        for kk in range(4):
            compute_code(kk)
            gdma.append(pltpu.async_copy(
                h_hbm.at[ipack.at[kk, pl.ds(0, _CH)]], hbuf.at[kk],
                gsem[kk]))
        for kk in range(4):
            gdma.append(pltpu.async_copy(
                etab_sh.at[codeb.at[kk]], ebuf.at[kk], isem[kk]))
        for d in gdma:
            d.wait()
        sdma = []
        for kk in range(4):
            if kk >= 2:
                sdma[kk - 2].wait()
            relu_msg(kk, kk % 2)
            sdma.append(pltpu.async_copy(
                mbuf.at[kk % 2], agg_sh.at[dstb.at[kk]], ssem[kk], add=True))
        sdma[2].wait()
        sdma[3].wait()

    plsc.subcore_barrier()

    nblk2 = 4 + jnp.where(s < 14, 1, 0)

    @pl.loop(0, nblk2)
    def _dblk(k):
        rb = (s + 16 * k) * _CH
        pltpu.sync_copy(agg_sh.at[pl.ds(rb, _CH)],
                        out_hbm.at[pl.ds(c * _N + rb, _CH)])

    @pl.when(s == 0)
    def _dtail():
        pltpu.sync_copy(agg_sh.at[pl.ds(78 * _CH, 16)],
                        out_hbm.at[pl.ds(c * _N + 78 * _CH, 16)])


def _tc_dense_layer(h, p, w1a, w1b, b1, w2, b2, gamma, beta):
    def body(h_ref, p_ref, w1a_ref, w1b_ref, b1_ref, w2_ref, b2_ref,
             g_ref, be_ref, o_ref):
        a_lo = h_ref[0:_N] + p_ref[0:_N]
        a_hi = h_ref[_N:2 * _N] + p_ref[_N:2 * _N]
        t = (jnp.dot(a_lo, w1a_ref[...], preferred_element_type=jnp.float32)
             + jnp.dot(a_hi, w1b_ref[...], preferred_element_type=jnp.float32))
        t = jnp.maximum(t + b1_ref[...], 0.0)
        u = jnp.dot(t, w2_ref[...], preferred_element_type=jnp.float32)
        u = u + b2_ref[...]
        mu = jnp.mean(u, axis=0, keepdims=True)
        var = jnp.mean((u - mu) * (u - mu), axis=0, keepdims=True)
        v = (u - mu) * lax.rsqrt(var + 1e-5) * g_ref[...] + be_ref[...]
        v = jnp.maximum(v, 0.0)
        o_ref[0:_N] = h_ref[0:_N] + v[:, 0:_HH]
        o_ref[_N:2 * _N] = h_ref[_N:2 * _N] + v[:, _HH:_H]

    return pl.pallas_call(
        body,
        out_shape=jax.ShapeDtypeStruct((2 * _N, _HH), jnp.float32),
    )(h, p, w1a, w1b, b1, w2, b2, gamma, beta)


def _tc_dense_pool_mlp(h, p, w1a, w1b, b1, w2, b2, gamma, beta,
                       batchT, w_lo, w_hi, mlp_b):
    def body(h_ref, p_ref, w1a_ref, w1b_ref, b1_ref, w2_ref, b2_ref,
             g_ref, be_ref, b_ref, wlo_ref, whi_ref, bias_ref, o_ref):
        a_lo = h_ref[0:_N] + p_ref[0:_N]
        a_hi = h_ref[_N:2 * _N] + p_ref[_N:2 * _N]
        t = (jnp.dot(a_lo, w1a_ref[...], preferred_element_type=jnp.float32)
             + jnp.dot(a_hi, w1b_ref[...], preferred_element_type=jnp.float32))
        t = jnp.maximum(t + b1_ref[...], 0.0)
        u = jnp.dot(t, w2_ref[...], preferred_element_type=jnp.float32)
        u = u + b2_ref[...]
        mu = jnp.mean(u, axis=0, keepdims=True)
        var = jnp.mean((u - mu) * (u - mu), axis=0, keepdims=True)
        v = (u - mu) * lax.rsqrt(var + 1e-5) * g_ref[...] + be_ref[...]
        v = jnp.maximum(v, 0.0)
        hf_lo = h_ref[0:_N] + v[:, 0:_HH]
        hf_hi = h_ref[_N:2 * _N] + v[:, _HH:_H]
        gid = lax.broadcasted_iota(jnp.int32, (_G, _N), 0)
        oh = (b_ref[...] == gid).astype(jnp.float32)
        sums_lo = jnp.dot(oh, hf_lo, preferred_element_type=jnp.float32)
        sums_hi = jnp.dot(oh, hf_hi, preferred_element_type=jnp.float32)
        inv = 1.0 / jnp.maximum(jnp.sum(oh, axis=1), 1.0)
        o_ref[...] = (jnp.dot(sums_lo * inv[:, None], wlo_ref[...],
                              preferred_element_type=jnp.float32)
                      + jnp.dot(sums_hi * inv[:, None], whi_ref[...],
                                preferred_element_type=jnp.float32)
                      + bias_ref[...])

    return pl.pallas_call(
        body,
        out_shape=jax.ShapeDtypeStruct((_G, _H), jnp.float32),
    )(h, p, w1a, w1b, b1, w2, b2, gamma, beta, batchT, w_lo, w_hi, mlp_b)


def kernel(x, edge_index, edge_attr, batch, atom_emb, bond_emb,
           W1, b1, W2, b2, bn_gamma, bn_beta, mlp_W, mlp_b):
    xpad = jnp.pad(x.astype(jnp.int32), ((0, _NPAD - _N), (0, 0)))
    xP = xpad.reshape(_NPAD // _CH, _CH, 9).transpose(0, 2, 1).reshape(-1)
    tab_full = atom_emb.reshape(9 * 128, _H)
    tab_cat = jnp.concatenate([tab_full[:, :_HH], tab_full[:, _HH:]], axis=0)
    src = jnp.pad(edge_index[0].astype(jnp.int32), (0, _EPAD - _E))
    dstp = jnp.pad(edge_index[1].astype(jnp.int32), (0, _EPAD - _E),
                   constant_values=_N)
    ea = jnp.pad(edge_attr.astype(jnp.int32), ((0, _EPAD - _E), (0, 0)))
    ipack = jnp.stack([src.reshape(_NCHUNKS_E, _CH),
                       ea[:, 0].reshape(_NCHUNKS_E, _CH),
                       ea[:, 1].reshape(_NCHUNKS_E, _CH),
                       ea[:, 2].reshape(_NCHUNKS_E, _CH)],
                      axis=1).reshape(-1)
    be_cat = jnp.concatenate([bond_emb[:, :, :_HH].reshape(-1),
                              bond_emb[:, :, _HH:].reshape(-1)])
    batchT = jnp.broadcast_to(batch.astype(jnp.int32)[None, :], (_G, _N))

    h0 = _sc_atom_encoder(xP, tab_cat)
    h = jnp.concatenate([h0[:_N], h0[_NPAD:_NPAD + _N]], axis=0)
    for i in range(2):
        p = _sc_edge_stage(h, ipack, dstp, be_cat)
        h = _tc_dense_layer(h, p, W1[i][:_HH], W1[i][_HH:],
                            b1[i].reshape(1, _H),
                            W2[i], b2[i].reshape(1, _H),
                            bn_gamma[i].reshape(1, _H),
                            bn_beta[i].reshape(1, _H))
    p = _sc_edge_stage(h, ipack, dstp, be_cat)
    return _tc_dense_pool_mlp(h, p, W1[2][:_HH], W1[2][_HH:],
                              b1[2].reshape(1, _H),
                              W2[2], b2[2].reshape(1, _H),
                              bn_gamma[2].reshape(1, _H),
                              bn_beta[2].reshape(1, _H),
                              batchT, mlp_W[:_HH], mlp_W[_HH:],
                              mlp_b.reshape(1, _H))


# speedup vs baseline: 1.5795x; 1.0069x over previous
import functools

import jax
import jax.numpy as jnp
from jax import lax
from jax.experimental import pallas as pl
from jax.experimental.pallas import tpu as pltpu
from jax.experimental.pallas import tpu_sc as plsc

_N = 10000
_NPAD = 10240
_E = 320000
_EPAD = 327680
_NAGG = 10016
_H = 128
_HH = 64
_G = 64
_CH = 128
_LANE = 16
_NCHUNKS_E = _EPAD // _CH
_TCH_E = _NCHUNKS_E // 16

_mesh = plsc.VectorSubcoreMesh(core_axis_name="c", subcore_axis_name="s")
_sc_params = pltpu.CompilerParams(use_tc_tiling_on_sc=False)


@functools.partial(pl.kernel,
                   out_type=jax.ShapeDtypeStruct((2 * _NPAD, _HH), jnp.float32),
                   mesh=_mesh,
                   compiler_params=_sc_params,
                   scratch_types=[
                       pltpu.VMEM((9 * _CH,), jnp.int32),
                       pltpu.VMEM((9, _CH, _HH), jnp.float32),
                       pltpu.VMEM((_CH, _HH), jnp.float32),
                       [pltpu.SemaphoreType.DMA] * 9,
                   ])
def _sc_atom_encoder(xP_hbm, tab_hbm, h0_hbm, ibuf, rows, acc, gsem):
    c = lax.axis_index("c")
    s = lax.axis_index("s")

    @pl.loop(0, _NPAD // _CH // 16)
    def _chunk(t):
        gc = s + 16 * t
        pltpu.sync_copy(xP_hbm.at[pl.ds(gc * 9 * _CH, 9 * _CH)], ibuf)

        @plsc.parallel_loop(0, 9 * _CH, _LANE)
        def _off(i):
            ibuf[pl.ds(i, _LANE)] = (ibuf[pl.ds(i, _LANE)]
                                     + c * (9 * _H) + (i // _CH) * _H)

        gdma = [pltpu.async_copy(tab_hbm.at[ibuf.at[pl.ds(f * _CH, _CH)]],
                                 rows.at[f], gsem[f]) for f in range(9)]
        for d in gdma:
            d.wait()

        @plsc.parallel_loop(0, _CH)
        def _acc(r):
            for j in range(0, _HH, _LANE):
                v = rows[0, r, pl.ds(j, _LANE)]
                for f in range(1, 9):
                    v = v + rows[f, r, pl.ds(j, _LANE)]
                acc[r, pl.ds(j, _LANE)] = v

        pltpu.sync_copy(acc, h0_hbm.at[pl.ds(c * _NPAD + gc * _CH, _CH)])


@functools.partial(pl.kernel,
                   out_type=jax.ShapeDtypeStruct((2 * _N, _HH), jnp.float32),
                   mesh=_mesh,
                   compiler_params=_sc_params,
                   scratch_types=[
                       pltpu.VMEM((4, 4 * _CH), jnp.int32),
                       pltpu.VMEM((4, _CH), jnp.int32),
                       pltpu.VMEM((4, _CH), jnp.int32),
                       pltpu.VMEM((3 * 8 * _HH,), jnp.float32),
                       pltpu.VMEM((4, _CH, _HH), jnp.float32),
                       pltpu.VMEM((4, _CH, _HH), jnp.float32),
                       pltpu.VMEM((2, _CH, _HH), jnp.float32),
                       pltpu.VMEM_SHARED((_NAGG, _HH), jnp.float32),
                       pltpu.VMEM_SHARED((512, _HH), jnp.float32),
                       [pltpu.SemaphoreType.DMA] * 4,
                       [pltpu.SemaphoreType.DMA] * 4,
                       [pltpu.SemaphoreType.DMA] * 4,
                   ])
def _sc_edge_stage(h_hbm, ip_hbm, dp_hbm, be_hbm, out_hbm,
                   ipack, dstb, codeb, bflat, hbuf, ebuf, mbuf,
                   agg_sh, etab_sh, isem, gsem, ssem):
    c = lax.axis_index("c")
    s = lax.axis_index("s")

    pltpu.sync_copy(be_hbm.at[pl.ds(c * (24 * _HH), 24 * _HH)], bflat)
    for rl in range(32):
        r = s * 32 + rl
        c0 = r // 64
        c1 = (r // 8) % 8
        c2 = r % 8
        for j in range(0, _HH, _LANE):
            ebuf[0, rl, pl.ds(j, _LANE)] = (
                bflat[pl.ds(c0 * _HH + j, _LANE)]
                + bflat[pl.ds(8 * _HH + c1 * _HH + j, _LANE)]
                + bflat[pl.ds(16 * _HH + c2 * _HH + j, _LANE)])
    pltpu.sync_copy(ebuf.at[0, pl.ds(0, 32)], etab_sh.at[pl.ds(s * 32, 32)])

    @plsc.parallel_loop(0, _CH)
    def _zrow(r):
        for j in range(0, _HH, _LANE):
            mbuf[0, r, pl.ds(j, _LANE)] = jnp.zeros((_LANE,), jnp.float32)

    nblk = 4 + jnp.where(s < 14, 1, 0)

    @pl.loop(0, nblk)
    def _zblk(k):
        pltpu.sync_copy(mbuf.at[0], agg_sh.at[pl.ds((s + 16 * k) * _CH, _CH)])

    @pl.when(s == 0)
    def _ztail():
        pltpu.sync_copy(mbuf.at[0, pl.ds(0, 32)],
                        agg_sh.at[pl.ds(78 * _CH, 32)])

    plsc.subcore_barrier()

    def issue_idx(t, b):
        gc = s + 16 * t
        pltpu.async_copy(ip_hbm.at[pl.ds(gc * 4 * _CH, 4 * _CH)],
                         ipack.at[b], isem[b])
        pltpu.async_copy(dp_hbm.at[pl.ds(gc * _CH, _CH)], dstb.at[b], isem[b])

    def wait_idx(t, b):
        gc = s + 16 * t
        pltpu.make_async_copy(ip_hbm.at[pl.ds(gc * 4 * _CH, 4 * _CH)],
                              ipack.at[b], isem[b]).wait()
        pltpu.make_async_copy(dp_hbm.at[pl.ds(gc * _CH, _CH)], dstb.at[b],
                              isem[b]).wait()

    def compute_code(b):
        @plsc.parallel_loop(0, _CH, _LANE)
        def _code(i):
            ipack[b, pl.ds(i, _LANE)] = ipack[b, pl.ds(i, _LANE)] + c * _N
            codeb[b, pl.ds(i, _LANE)] = (
                ipack[b, pl.ds(_CH + i, _LANE)] * 64
                + ipack[b, pl.ds(2 * _CH + i, _LANE)] * 8
                + ipack[b, pl.ds(3 * _CH + i, _LANE)])

    def relu_msg(b, mb):
        @plsc.parallel_loop(0, _CH, 1, unroll=4)
        def _relu(r):
            for j in range(0, _HH, _LANE):
                hv = hbuf[b, r, pl.ds(j, _LANE)]
                ev = ebuf[b, r, pl.ds(j, _LANE)]
                mbuf[mb, r, pl.ds(j, _LANE)] = jnp.maximum(hv + ev, 0.0)

    @pl.loop(0, _TCH_E // 4)
    def _quad(qq):
        idma = []
        for kk in range(4):
            gc = s + 16 * (4 * qq + kk)
            idma.append(pltpu.async_copy(
                ip_hbm.at[pl.ds(gc * 4 * _CH, 4 * _CH)], ipack.at[kk],
                isem[kk]))
            idma.append(pltpu.async_copy(
                dp_hbm.at[pl.ds(gc * _CH, _CH)], dstb.at[kk], isem[kk]))
        for d in idma:
            d.wait()
        gdma = []
        for kk in range(4):
            compute_code(kk)
            gdma.append(pltpu.async_copy(
                h_hbm.at[ipack.at[kk, pl.ds(0, _CH)]], hbuf.at[kk],
                gsem[kk]))
        for kk in range(4):
            gdma.append(pltpu.async_copy(
                etab_sh.at[codeb.at[kk]], ebuf.at[kk], isem[kk]))
        for d in gdma:
            d.wait()
        sdma = []
        for kk in range(4):
            if kk >= 2:
                sdma[kk - 2].wait()
            relu_msg(kk, kk % 2)
            sdma.append(pltpu.async_copy(
                mbuf.at[kk % 2], agg_sh.at[dstb.at[kk]], ssem[kk], add=True))
        sdma[2].wait()
        sdma[3].wait()

    plsc.subcore_barrier()

    nblk2 = 4 + jnp.where(s < 14, 1, 0)

    @pl.loop(0, nblk2)
    def _dblk(k):
        rb = (s + 16 * k) * _CH
        pltpu.sync_copy(agg_sh.at[pl.ds(rb, _CH)],
                        out_hbm.at[pl.ds(c * _N + rb, _CH)])

    @pl.when(s == 0)
    def _dtail():
        pltpu.sync_copy(agg_sh.at[pl.ds(78 * _CH, 16)],
                        out_hbm.at[pl.ds(c * _N + 78 * _CH, 16)])


def _tc_dense_layer(h, p, w1a, w1b, b1, w2, b2, gamma, beta):
    def body(h_ref, p_ref, w1a_ref, w1b_ref, b1_ref, w2_ref, b2_ref,
             g_ref, be_ref, o_ref):
        a_lo = h_ref[0:_N] + p_ref[0:_N]
        a_hi = h_ref[_N:2 * _N] + p_ref[_N:2 * _N]
        t = (jnp.dot(a_lo, w1a_ref[...], preferred_element_type=jnp.float32)
             + jnp.dot(a_hi, w1b_ref[...], preferred_element_type=jnp.float32))
        t = jnp.maximum(t + b1_ref[...], 0.0)
        u = jnp.dot(t, w2_ref[...], preferred_element_type=jnp.float32)
        u = u + b2_ref[...]
        mu = jnp.mean(u, axis=0, keepdims=True)
        var = jnp.mean((u - mu) * (u - mu), axis=0, keepdims=True)
        v = (u - mu) * lax.rsqrt(var + 1e-5) * g_ref[...] + be_ref[...]
        v = jnp.maximum(v, 0.0)
        o_ref[0:_N] = h_ref[0:_N] + v[:, 0:_HH]
        o_ref[_N:2 * _N] = h_ref[_N:2 * _N] + v[:, _HH:_H]

    return pl.pallas_call(
        body,
        out_shape=jax.ShapeDtypeStruct((2 * _N, _HH), jnp.float32),
    )(h, p, w1a, w1b, b1, w2, b2, gamma, beta)


def _tc_dense_pool_mlp(h, p, w1a, w1b, b1, w2, b2, gamma, beta,
                       batchT, w_lo, w_hi, mlp_b):
    def body(h_ref, p_ref, w1a_ref, w1b_ref, b1_ref, w2_ref, b2_ref,
             g_ref, be_ref, b_ref, wlo_ref, whi_ref, bias_ref, o_ref):
        a_lo = h_ref[0:_N] + p_ref[0:_N]
        a_hi = h_ref[_N:2 * _N] + p_ref[_N:2 * _N]
        t = (jnp.dot(a_lo, w1a_ref[...], preferred_element_type=jnp.float32)
             + jnp.dot(a_hi, w1b_ref[...], preferred_element_type=jnp.float32))
        t = jnp.maximum(t + b1_ref[...], 0.0)
        u = jnp.dot(t, w2_ref[...], preferred_element_type=jnp.float32)
        u = u + b2_ref[...]
        mu = jnp.mean(u, axis=0, keepdims=True)
        var = jnp.mean((u - mu) * (u - mu), axis=0, keepdims=True)
        v = (u - mu) * lax.rsqrt(var + 1e-5) * g_ref[...] + be_ref[...]
        v = jnp.maximum(v, 0.0)
        hf_lo = h_ref[0:_N] + v[:, 0:_HH]
        hf_hi = h_ref[_N:2 * _N] + v[:, _HH:_H]
        gid = lax.broadcasted_iota(jnp.int32, (_G, _N), 0)
        oh = (b_ref[...] == gid).astype(jnp.float32)
        sums_lo = jnp.dot(oh, hf_lo, preferred_element_type=jnp.float32)
        sums_hi = jnp.dot(oh, hf_hi, preferred_element_type=jnp.float32)
        inv = 1.0 / jnp.maximum(jnp.sum(oh, axis=1), 1.0)
        o_ref[...] = (jnp.dot(sums_lo * inv[:, None], wlo_ref[...],
                              preferred_element_type=jnp.float32)
                      + jnp.dot(sums_hi * inv[:, None], whi_ref[...],
                                preferred_element_type=jnp.float32)
                      + bias_ref[...])

    return pl.pallas_call(
        body,
        out_shape=jax.ShapeDtypeStruct((_G, _H), jnp.float32),
    )(h, p, w1a, w1b, b1, w2, b2, gamma, beta, batchT, w_lo, w_hi, mlp_b)


def kernel(x, edge_index, edge_attr, batch, atom_emb, bond_emb,
           W1, b1, W2, b2, bn_gamma, bn_beta, mlp_W, mlp_b):
    xpad = jnp.pad(x.astype(jnp.int32), ((0, _NPAD - _N), (0, 0)))
    xP = xpad.reshape(_NPAD // _CH, _CH, 9).transpose(0, 2, 1).reshape(-1)
    tab_full = atom_emb.reshape(9 * 128, _H)
    tab_cat = jnp.concatenate([tab_full[:, :_HH], tab_full[:, _HH:]], axis=0)
    src = jnp.pad(edge_index[0].astype(jnp.int32), (0, _EPAD - _E))
    dstp = jnp.pad(edge_index[1].astype(jnp.int32), (0, _EPAD - _E),
                   constant_values=_N)
    ea = jnp.pad(edge_attr.astype(jnp.int32), ((0, _EPAD - _E), (0, 0)))
    ipack = jnp.stack([src.reshape(_NCHUNKS_E, _CH),
                       ea[:, 0].reshape(_NCHUNKS_E, _CH),
                       ea[:, 1].reshape(_NCHUNKS_E, _CH),
                       ea[:, 2].reshape(_NCHUNKS_E, _CH)],
                      axis=1).reshape(-1)
    be_cat = jnp.concatenate([bond_emb[:, :, :_HH].reshape(-1),
                              bond_emb[:, :, _HH:].reshape(-1)])
    batchT = jnp.broadcast_to(batch.astype(jnp.int32)[None, :], (_G, _N))

    h0 = _sc_atom_encoder(xP, tab_cat)
    h = jnp.concatenate([h0[:_N], h0[_NPAD:_NPAD + _N]], axis=0)
    for i in range(2):
        p = _sc_edge_stage(h, ipack, dstp, be_cat)
        h = _tc_dense_layer(h, p, W1[i][:_HH], W1[i][_HH:],
                            b1[i].reshape(1, _H),
                            W2[i], b2[i].reshape(1, _H),
                            bn_gamma[i].reshape(1, _H),
                            bn_beta[i].reshape(1, _H))
    p = _sc_edge_stage(h, ipack, dstp, be_cat)
    return _tc_dense_pool_mlp(h, p, W1[2][:_HH], W1[2][_HH:],
                              b1[2].reshape(1, _H),
                              W2[2], b2[2].reshape(1, _H),
                              bn_gamma[2].reshape(1, _H),
                              bn_beta[2].reshape(1, _H),
                              batchT, mlp_W[:_HH], mlp_W[_HH:],
                              mlp_b.reshape(1, _H))
